# baseline (speedup 1.0000x reference)
import functools

import jax
import jax.numpy as jnp
from jax import lax
from jax.experimental import pallas as pl
from jax.experimental.pallas import tpu as pltpu
from jax.experimental.pallas import tpu_sc as plsc

B = 16384
D = 64
NC = 2
NS = 16
L = 16
NW = NC * NS
BPW = B // NW
GCH = 128
NG = BPW // GCH
W = 2 * D


@functools.partial(
    pl.kernel,
    out_type=jax.ShapeDtypeStruct((B, D), jnp.float32),
    mesh=plsc.VectorSubcoreMesh(core_axis_name="c", subcore_axis_name="s",
                                num_cores=NC, num_subcores=NS),
    scratch_types=[
        pltpu.VMEM((NG, GCH), jnp.int32),
        pltpu.VMEM((BPW,), jnp.int32),
        pltpu.VMEM((BPW, W), jnp.float32),
        pltpu.VMEM((BPW, D), jnp.float32),
        pltpu.SemaphoreType.DMA,
        pltpu.SemaphoreType.DMA,
    ],
    compiler_params=pltpu.CompilerParams(use_tc_tiling_on_sc=False,
                                         needs_layout_passes=False),
)
def _emb_mul(idx_hbm, off_hbm, lat_hbm, tab_hbm, out_hbm,
             idx_v, off_v, rows_v, lat_v, gsem, lsem):
    wid = lax.axis_index("s") * NC + lax.axis_index("c")
    base = wid * BPW

    pltpu.sync_copy(idx_hbm.at[wid], idx_v)
    off_cp = pltpu.async_copy(off_hbm.at[pl.ds(base, BPW)], off_v, lsem)
    lat_cp = pltpu.async_copy(lat_hbm.at[pl.ds(base, BPW)], lat_v, lsem)
    gathers = [
        pltpu.async_copy(tab_hbm.at[idx_v.at[g]],
                         rows_v.at[pl.ds(g * GCH, GCH)], gsem)
        for g in range(NG)
    ]
    off_cp.wait()
    lat_cp.wait()
    for cp in gathers:
        cp.wait()

    lanes = lax.iota(jnp.int32, L)

    def row(i, carry):
        isp = jnp.full((L,), i, jnp.int32)
        hsp = plsc.load_gather(off_v, [isp])
        for j in range(D // L):
            col = hsp + (j * L + lanes)
            v = plsc.load_gather(rows_v, [isp, col])
            s = pl.ds(j * L, L)
            lat_v[i, s] = v * lat_v[i, s]
        return carry

    lax.fori_loop(0, BPW, row, 0)

    pltpu.sync_copy(lat_v, out_hbm.at[pl.ds(base, BPW)])


def kernel(label, latent, emb_table):
    lab = label.astype(jnp.int32)
    tab2 = emb_table.reshape(500_000, W)
    idx = (lab >> 1).reshape(NW, NG, GCH)
    off = (lab & 1) * D
    return _emb_mul(idx, off, latent, tab2)


# speedup vs baseline: 2.9299x; 2.9299x over previous
import functools

import jax
import jax.numpy as jnp
from jax import lax
from jax.experimental import pallas as pl
from jax.experimental.pallas import tpu as pltpu
from jax.experimental.pallas import tpu_sc as plsc

B = 16384
D = 64
NC = 2
NS = 16
L = 16
NW = NC * NS
BPW = B // NW
NBLK = BPW // L
TW = 128
NSLOT = 4


@functools.partial(
    pl.kernel,
    out_type=jax.ShapeDtypeStruct((B, D), jnp.float32),
    mesh=plsc.VectorSubcoreMesh(core_axis_name="c", subcore_axis_name="s",
                                num_cores=NC, num_subcores=NS),
    scratch_types=[
        pltpu.VMEM((BPW + L,), jnp.int32),
        pltpu.VMEM((BPW,), jnp.int32),
        pltpu.VMEM((NSLOT, D, TW), jnp.float32),
        pltpu.VMEM((BPW, D), jnp.float32),
        pltpu.SemaphoreType.DMA,
        pltpu.SemaphoreType.DMA,
    ],
    compiler_params=pltpu.CompilerParams(needs_layout_passes=False),
)
def _emb_mul(b128_hbm, lane_hbm, lat_hbm, tab_hbm, out_hbm,
             b128_v, lane_v, ring_v, lat_v, gsem, lsem):
    wid = lax.axis_index("s") * NC + lax.axis_index("c")
    base = wid * BPW

    pltpu.sync_copy(b128_hbm.at[pl.ds(base, BPW)], b128_v.at[pl.ds(0, BPW)])
    pltpu.sync_copy(lane_hbm.at[pl.ds(base, BPW)], lane_v)
    lat_cp = pltpu.async_copy(lat_hbm.at[pl.ds(base, BPW)], lat_v, lsem)

    lanes = lax.iota(jnp.int32, L)

    def fire(b128_scalar, slot):
        pltpu.async_copy(
            tab_hbm.at[:, pl.ds(pl.multiple_of(b128_scalar, TW), TW)],
            ring_v.at[slot], gsem)

    b0 = b128_v[pl.ds(0, L)]
    for t in range(NSLOT):
        fire(b0[t], t)
    lat_cp.wait()

    def block(k, carry):
        i0 = k * L
        bcur = b128_v[pl.ds(i0, L)]
        bnext = b128_v[pl.ds(i0 + L, L)]
        lnb = lane_v[pl.ds(i0, L)]
        for t in range(L):
            i = i0 + t
            slot = t % NSLOT
            pltpu.make_async_copy(tab_hbm.at[:, pl.ds(0, TW)],
                                  ring_v.at[slot], gsem).wait()
            lsp = jnp.full((L,), lnb[t], jnp.int32)
            ssp = jnp.full((L,), slot, jnp.int32)
            for j in range(D // L):
                col = j * L + lanes
                v = plsc.load_gather(ring_v, [ssp, col, lsp])
                s = pl.ds(j * L, L)
                lat_v[i, s] = v * lat_v[i, s]

            bref = bcur[t + NSLOT] if t + NSLOT < L else bnext[t - (L - NSLOT)]

            @pl.when(i < BPW - NSLOT)
            def _():
                fire(bref, slot)

        return carry

    lax.fori_loop(0, NBLK, block, 0)

    pltpu.sync_copy(lat_v, out_hbm.at[pl.ds(base, BPW)])


def kernel(label, latent, emb_table):
    lab = label.astype(jnp.int32)
    b128 = lab & ~(TW - 1)
    lane = lab & (TW - 1)
    return _emb_mul(b128, lane, latent, emb_table.T)
